# Initial kernel scaffold; baseline (speedup 1.0000x reference)
import jax
import jax.numpy as jnp
from jax.experimental import pallas as pl


def kernel(data, segment_ids):
    raise NotImplementedError("write your pallas kernel here")


# speedup vs baseline: 1.5120x; 1.5120x over previous
import functools

import jax
import jax.numpy as jnp
from jax import lax
from jax.experimental import pallas as pl
from jax.experimental.pallas import tpu as pltpu
from jax.experimental.pallas import tpu_sc as plsc

NUM_SEG = 16
TOTAL_TOK = 32768
D = 256
L = 16

NC = 2
NS = 16
NW = NC * NS
TOK_PER_W = TOTAL_TOK // NW
CHUNK = 128
NCHUNK = TOK_PER_W // CHUNK


def _sc_partial_sums(data, ids, zeros):
  mesh = plsc.VectorSubcoreMesh(core_axis_name="c", subcore_axis_name="s")

  @functools.partial(
      pl.kernel,
      out_type=jax.ShapeDtypeStruct((NC, NUM_SEG, D), jnp.float32),
      mesh=mesh,
      scratch_types=[
          pltpu.VMEM((CHUNK, D), jnp.float32),
          pltpu.VMEM((CHUNK,), jnp.int32),
          pltpu.VMEM((NUM_SEG, D), jnp.float32),
          pltpu.VMEM((NS, D), jnp.float32),
          pltpu.VMEM_SHARED((NS, NUM_SEG, D), jnp.float32),
      ],
  )
  def k(data_hbm, ids_hbm, zeros_hbm, out_hbm, dbuf, ibuf, acc, tbuf,
        shared):
    cid = lax.axis_index("c")
    sid = lax.axis_index("s")
    wid = cid * NS + sid

    pltpu.sync_copy(zeros_hbm, acc)

    for ch in range(NCHUNK):
      base = wid * TOK_PER_W + ch * CHUNK
      pltpu.sync_copy(data_hbm.at[pl.ds(base, CHUNK)], dbuf)
      pltpu.sync_copy(ids_hbm.at[pl.ds(base, CHUNK)], ibuf)

      def group_body(g):
        segs = ibuf[pl.ds(g * L, L)]
        for j in range(L):
          seg = segs[j]
          t = g * L + j
          for d in range(D // L):
            sl = pl.ds(d * L, L)
            plsc.addupdate(acc.at[seg, sl], dbuf[t, sl])

      pl.loop(0, CHUNK // L)(group_body)

    pltpu.sync_copy(acc, shared.at[sid])
    plsc.subcore_barrier()
    pltpu.sync_copy(shared.at[:, sid], tbuf)
    for d in range(D // L):
      sl = pl.ds(d * L, L)
      v = tbuf[0, sl]
      for t in range(1, NS):
        v = v + tbuf[t, sl]
      acc[0, sl] = v
    pltpu.sync_copy(acc.at[0], out_hbm.at[cid, sid])

  return k(data, ids, zeros)


def _merge_body(p_ref, o_ref):
  o_ref[...] = p_ref[0] + p_ref[1]


def _merge(partials):
  return pl.pallas_call(
      _merge_body,
      out_shape=jax.ShapeDtypeStruct((NUM_SEG, D), jnp.float32),
  )(partials)


@jax.jit
def kernel(data, segment_ids):
  ids = segment_ids.astype(jnp.int32)
  zeros = jnp.zeros((NUM_SEG, D), jnp.float32)
  partials = _sc_partial_sums(data, ids, zeros)
  return _merge(partials)


# speedup vs baseline: 1.9071x; 1.2613x over previous
import functools

import jax
import jax.numpy as jnp
from jax import lax
from jax.experimental import pallas as pl
from jax.experimental.pallas import tpu as pltpu
from jax.experimental.pallas import tpu_sc as plsc

NUM_SEG = 16
TOTAL_TOK = 32768
D = 256
L = 16

NC = 2
NS = 16
COLS = D // NC
CW = COLS // L

TOK_PER_TILE = TOTAL_TOK // NS
CHUNK = 128
NCHUNK = TOK_PER_TILE // CHUNK
GROUPS = CHUNK // L
NBUF = 2


def _sc_segment_sum(data3, ids):
  mesh = plsc.VectorSubcoreMesh(core_axis_name="c", subcore_axis_name="s")

  @functools.partial(
      pl.kernel,
      out_type=jax.ShapeDtypeStruct((NUM_SEG, NC, COLS), jnp.float32),
      mesh=mesh,
      scratch_types=[
          pltpu.VMEM((NBUF, CHUNK, COLS), jnp.float32),
          pltpu.VMEM((NBUF, CHUNK), jnp.int32),
          pltpu.VMEM((NUM_SEG, COLS), jnp.float32),
          pltpu.VMEM((NS, COLS), jnp.float32),
          pltpu.VMEM((COLS,), jnp.float32),
          pltpu.VMEM_SHARED((NS, NUM_SEG, COLS), jnp.float32),
          pltpu.SemaphoreType.DMA((NBUF,)),
      ],
  )
  def k(data_hbm, ids_hbm, out_hbm, dbuf, ibuf, acc, tbuf, obuf, shared,
        sems):
    cid = lax.axis_index("c")
    sid = lax.axis_index("s")
    row0 = sid * TOK_PER_TILE

    zero = jnp.zeros((L,), jnp.float32)
    for s in range(NUM_SEG):
      for d in range(CW):
        acc[s, pl.ds(d * L, L)] = zero

    def start(ch, b):
      base = row0 + ch * CHUNK
      pltpu.async_copy(
          data_hbm.at[pl.ds(base, CHUNK), cid], dbuf.at[b], sems.at[b]
      )
      pltpu.async_copy(ids_hbm.at[pl.ds(base, CHUNK)], ibuf.at[b],
                       sems.at[b])

    def drain(b):
      pltpu.make_async_copy(
          data_hbm.at[pl.ds(0, CHUNK), cid], dbuf.at[b], sems.at[b]
      ).wait()
      pltpu.make_async_copy(
          ids_hbm.at[pl.ds(0, CHUNK)], ibuf.at[b], sems.at[b]
      ).wait()

    def compute(b):
      def group_body(g):
        segs = ibuf[b, pl.ds(g * L, L)]
        s_first = segs[0]
        s_last = segs[L - 1]

        @pl.when(s_first == s_last)
        def _uniform():
          for d in range(CW):
            sl = pl.ds(d * L, L)
            v = dbuf[b, g * L, sl]
            for j in range(1, L):
              v = v + dbuf[b, g * L + j, sl]
            plsc.addupdate(acc.at[s_first, sl], v)

        @pl.when(s_first != s_last)
        def _mixed():
          for j in range(L):
            seg = segs[j]
            for d in range(CW):
              sl = pl.ds(d * L, L)
              plsc.addupdate(acc.at[seg, sl], dbuf[b, g * L + j, sl])

      pl.loop(0, GROUPS)(group_body)

    for b in range(NBUF):
      start(b, b)

    def chunk_body(i):
      ch = i * NBUF
      for b in range(NBUF):
        drain(b)
        compute(b)
        this_ch = ch + b

        @pl.when(this_ch + NBUF < NCHUNK)
        def _next():
          start(this_ch + NBUF, b)

    pl.loop(0, NCHUNK // NBUF)(chunk_body)

    pltpu.sync_copy(acc, shared.at[sid])
    plsc.subcore_barrier()
    pltpu.sync_copy(shared.at[:, sid], tbuf)
    for d in range(CW):
      sl = pl.ds(d * L, L)
      v = tbuf[0, sl]
      for t in range(1, NS):
        v = v + tbuf[t, sl]
      obuf[sl] = v
    pltpu.sync_copy(obuf, out_hbm.at[sid, cid])

  return k(data3, ids)


@jax.jit
def kernel(data, segment_ids):
  ids = segment_ids.astype(jnp.int32)
  data3 = data.reshape(TOTAL_TOK, NC, COLS)
  out3 = _sc_segment_sum(data3, ids)
  return out3.reshape(NUM_SEG, D)


# speedup vs baseline: 3.3292x; 1.7456x over previous
import functools

import jax
import jax.numpy as jnp
from jax import lax
from jax.experimental import pallas as pl
from jax.experimental.pallas import tpu as pltpu
from jax.experimental.pallas import tpu_sc as plsc

NUM_SEG = 16
TOTAL_TOK = 32768
D = 256
L = 16

NC = 2
NS = 16
COLS = D // NC
CW = COLS // L

TOK_PER_TILE = TOTAL_TOK // NS
CHUNK = 128
NCHUNK = TOK_PER_TILE // CHUNK
GROUPS = CHUNK // L
NBUF = 2


def _sc_segment_sum(data, ids):
  mesh = plsc.VectorSubcoreMesh(core_axis_name="c", subcore_axis_name="s")

  @functools.partial(
      pl.kernel,
      out_type=jax.ShapeDtypeStruct((NUM_SEG, D), jnp.float32),
      mesh=mesh,
      scratch_types=[
          pltpu.VMEM((NBUF, CHUNK, COLS), jnp.float32),
          pltpu.VMEM((NBUF, CHUNK), jnp.int32),
          pltpu.VMEM((NUM_SEG, COLS), jnp.float32),
          pltpu.VMEM((NS, COLS), jnp.float32),
          pltpu.VMEM((COLS,), jnp.float32),
          pltpu.VMEM_SHARED((NS, NUM_SEG, COLS), jnp.float32),
          pltpu.SemaphoreType.DMA((NBUF,)),
      ],
  )
  def k(data_hbm, ids_hbm, out_hbm, dbuf, ibuf, acc, tbuf, obuf, shared,
        sems):
    cid = lax.axis_index("c")
    sid = lax.axis_index("s")
    row0 = sid * TOK_PER_TILE

    zero = jnp.zeros((L,), jnp.float32)
    for s in range(NUM_SEG):
      for d in range(CW):
        acc[s, pl.ds(d * L, L)] = zero

    col0 = cid * COLS

    def start(ch, b):
      base = row0 + ch * CHUNK
      pltpu.async_copy(
          data_hbm.at[pl.ds(base, CHUNK), pl.ds(col0, COLS)], dbuf.at[b],
          sems.at[b]
      )
      pltpu.async_copy(ids_hbm.at[pl.ds(base, CHUNK)], ibuf.at[b],
                       sems.at[b])

    def drain(b):
      pltpu.make_async_copy(
          data_hbm.at[pl.ds(0, CHUNK), pl.ds(col0, COLS)], dbuf.at[b],
          sems.at[b]
      ).wait()
      pltpu.make_async_copy(
          ids_hbm.at[pl.ds(0, CHUNK)], ibuf.at[b], sems.at[b]
      ).wait()

    def compute(b):
      def group_body(g):
        segs = ibuf[b, pl.ds(g * L, L)]
        s_first = segs[0]
        s_last = segs[L - 1]

        @pl.when(s_first == s_last)
        def _uniform():
          for d in range(CW):
            sl = pl.ds(d * L, L)
            v = dbuf[b, g * L, sl]
            for j in range(1, L):
              v = v + dbuf[b, g * L + j, sl]
            plsc.addupdate(acc.at[s_first, sl], v)

        @pl.when(s_first != s_last)
        def _mixed():
          for j in range(L):
            seg = segs[j]
            for d in range(CW):
              sl = pl.ds(d * L, L)
              plsc.addupdate(acc.at[seg, sl], dbuf[b, g * L + j, sl])

      pl.loop(0, GROUPS)(group_body)

    for b in range(NBUF):
      start(b, b)

    def chunk_body(i):
      ch = i * NBUF
      for b in range(NBUF):
        drain(b)
        compute(b)
        this_ch = ch + b

        @pl.when(this_ch + NBUF < NCHUNK)
        def _next():
          start(this_ch + NBUF, b)

    pl.loop(0, NCHUNK // NBUF)(chunk_body)

    pltpu.sync_copy(acc, shared.at[sid])
    plsc.subcore_barrier()
    pltpu.sync_copy(shared.at[:, sid], tbuf)
    for d in range(CW):
      sl = pl.ds(d * L, L)
      v = tbuf[0, sl]
      for t in range(1, NS):
        v = v + tbuf[t, sl]
      obuf[sl] = v
    pltpu.sync_copy(obuf, out_hbm.at[sid, pl.ds(col0, COLS)])

  return k(data, ids)


@jax.jit
def kernel(data, segment_ids):
  ids = segment_ids.astype(jnp.int32)
  return _sc_segment_sum(data, ids)
